# baseline (speedup 1.0000x reference)
import functools

import jax
import jax.numpy as jnp
from jax import lax
from jax.experimental import pallas as pl
from jax.experimental.pallas import tpu as pltpu
from jax.experimental.pallas import tpu_sc as plsc

N_NODES = 10000
NP = 10240
NACC = 10368
D = 128
N_EDGES = 320000

NC = 2
NS = 16
NW = NC * NS
CHUNK = 128
GROUP = 8
ROWS_PER_TILE = NACC // NS

EP1 = 655360
CPT1 = EP1 // (NW * CHUNK)
EP2 = 327680
CPT2 = EP2 // (NW * CHUNK)

_SELU_ALPHA = 1.6732632423543772
_SELU_SCALE = 1.0507009873554805


def _seg_body(chunks_per_tile, table, src2d, dst2d, zrows, out,
              sidx, didx, rows0, rows1, gsem0, gsem1, ssem0, ssem1, acc):
    c = lax.axis_index("c")
    s = lax.axis_index("s")
    wid = c * NS + s
    base = wid * chunks_per_tile
    rows = (rows0, rows1)
    gsems = (gsem0, gsem1)
    ssems = (ssem0, ssem1)

    pltpu.sync_copy(zrows, acc.at[pl.ds(s * ROWS_PER_TILE, ROWS_PER_TILE)])

    plsc.subcore_barrier()

    def group_body(g, carry):
        gsl = pl.ds(base + g * GROUP, GROUP)
        pltpu.sync_copy(src2d.at[gsl], sidx)
        pltpu.sync_copy(dst2d.at[gsl], didx)

        gd = [None, None]
        sd = [None, None]
        for i in range(GROUP):
            b = i % 2
            if sd[b] is not None:
                sd[b].wait()
            gd[b] = pltpu.async_copy(table.at[sidx.at[i]], rows[b], gsems[b])
            if i > 0:
                pb = 1 - b
                gd[pb].wait()
                sd[pb] = pltpu.async_copy(
                    rows[pb], acc.at[didx.at[i - 1]], ssems[pb], add=True)
        lb = (GROUP - 1) % 2
        gd[lb].wait()
        sd[lb] = pltpu.async_copy(
            rows[lb], acc.at[didx.at[GROUP - 1]], ssems[lb], add=True)
        sd[0].wait()
        sd[1].wait()
        return carry

    lax.fori_loop(0, chunks_per_tile // GROUP, group_body, 0)

    plsc.subcore_barrier()

    sl = pl.ds(s * ROWS_PER_TILE, ROWS_PER_TILE)
    pltpu.sync_copy(acc.at[sl], out.at[c].at[sl])


def _make_seg_kernel(chunks_per_tile):
    return pl.kernel(
        functools.partial(_seg_body, chunks_per_tile),
        out_type=jax.ShapeDtypeStruct((NC, NACC, D), jnp.float32),
        mesh=plsc.VectorSubcoreMesh(core_axis_name="c", subcore_axis_name="s"),
        scratch_types=[
            pltpu.VMEM((GROUP, CHUNK), jnp.int32),
            pltpu.VMEM((GROUP, CHUNK), jnp.int32),
            pltpu.VMEM((CHUNK, D), jnp.float32),
            pltpu.VMEM((CHUNK, D), jnp.float32),
            pltpu.SemaphoreType.DMA,
            pltpu.SemaphoreType.DMA,
            pltpu.SemaphoreType.DMA,
            pltpu.SemaphoreType.DMA,
            pltpu.VMEM_SHARED((NACC, D), jnp.float32),
        ],
    )


_seg_deg = _make_seg_kernel(CPT1)
_seg = _make_seg_kernel(CPT2)


def _selu(v):
    return _SELU_SCALE * jnp.where(v > 0, v, _SELU_ALPHA * (jnp.exp(v) - 1.0))


def _tc_body(activation, p_ref, deg_ref, w_ref, b_ref, o_ref):
    p = p_ref[0] + p_ref[1]
    dinv = 1.0 / jnp.maximum(deg_ref[...], 1.0)
    v = jnp.dot(p * dinv, w_ref[...], preferred_element_type=jnp.float32)
    v = v + b_ref[...]
    if activation:
        v = _selu(v)
    o_ref[...] = v


RB = 1024


def _make_tc_kernel(activation):
    return pl.pallas_call(
        functools.partial(_tc_body, activation),
        grid=(NP // RB,),
        in_specs=[
            pl.BlockSpec((NC, RB, D), lambda i: (0, i, 0)),
            pl.BlockSpec((RB, 1), lambda i: (i, 0)),
            pl.BlockSpec((D, D), lambda i: (0, 0)),
            pl.BlockSpec((1, D), lambda i: (0, 0)),
        ],
        out_specs=pl.BlockSpec((RB, D), lambda i: (i, 0)),
        out_shape=jax.ShapeDtypeStruct((NP, D), jnp.float32),
    )


_tc_act = _make_tc_kernel(True)
_tc_lin = _make_tc_kernel(False)


def kernel(x, edge_index, W1, b1, W2, b2):
    ei = edge_index.astype(jnp.int32)
    src, dst = ei[0], ei[1]

    oh_src = N_NODES + (dst & 127)
    deg_dst = NP + lax.shift_right_logical(dst, 7)
    half = N_EDGES // 2
    padn = (EP1 - 2 * N_EDGES) // 2
    spad = jnp.zeros((padn,), jnp.int32)
    dpad = jnp.full((padn,), NACC - 1, jnp.int32)
    s1 = jnp.concatenate(
        [src[:half], oh_src[:half], spad, src[half:], oh_src[half:], spad])
    d1 = jnp.concatenate(
        [dst[:half], deg_dst[:half], dpad, dst[half:], deg_dst[half:], dpad])
    src2d_1 = s1.reshape(-1, CHUNK)
    dst2d_1 = d1.reshape(-1, CHUNK)

    s2 = jnp.concatenate([src, jnp.zeros((EP2 - N_EDGES,), jnp.int32)])
    d2 = jnp.concatenate(
        [dst, jnp.full((EP2 - N_EDGES,), NACC - 1, jnp.int32)])
    src2d_2 = s2.reshape(-1, CHUNK)
    dst2d_2 = d2.reshape(-1, CHUNK)

    xt = jnp.concatenate([x, jnp.eye(D, dtype=jnp.float32)], axis=0)
    z = jnp.zeros((ROWS_PER_TILE, D), jnp.float32)

    p = _seg_deg(xt, src2d_1, dst2d_1, z)
    deg = (p[0, NP:NP + 80] + p[1, NP:NP + 80]).reshape(NP, 1)
    h = _tc_act(p, deg, W1, b1.reshape(1, D))
    q = _seg(h, src2d_2, dst2d_2, z)
    out = _tc_lin(q, deg, W2, b2.reshape(1, D))
    return out[:N_NODES]


# speedup vs baseline: 1.2005x; 1.2005x over previous
import functools

import jax
import jax.numpy as jnp
from jax import lax
from jax.experimental import pallas as pl
from jax.experimental.pallas import tpu as pltpu
from jax.experimental.pallas import tpu_sc as plsc

N_NODES = 10000
NP = 10240
NACC = 10368
D = 128
N_EDGES = 320000

NC = 2
NS = 16
NW = NC * NS
CHUNK = 128
GROUP = 8
ROWS_PER_TILE = NACC // NS

PHASE_ROWS = 2560
PHASE_CPT = PHASE_ROWS // NW
PHASE_E = PHASE_ROWS * CHUNK

_SELU_ALPHA = 1.6732632423543772
_SELU_SCALE = 1.0507009873554805


def _run_phase(table, src2d, dst2d, acc, sidx, didx, rows, gsems, ssems,
               base, nchunks):
    def group_body(g, carry):
        gsl = pl.ds(base + g * GROUP, GROUP)
        pltpu.sync_copy(src2d.at[gsl], sidx)
        pltpu.sync_copy(dst2d.at[gsl], didx)

        gd = [None, None]
        sd = [None, None]
        for i in range(GROUP):
            b = i % 2
            if sd[b] is not None:
                sd[b].wait()
            gd[b] = pltpu.async_copy(table.at[sidx.at[i]], rows[b], gsems[b])
            if i > 0:
                pb = 1 - b
                gd[pb].wait()
                sd[pb] = pltpu.async_copy(
                    rows[pb], acc.at[didx.at[i - 1]], ssems[pb], add=True)
        lb = (GROUP - 1) % 2
        gd[lb].wait()
        sd[lb] = pltpu.async_copy(
            rows[lb], acc.at[didx.at[GROUP - 1]], ssems[lb], add=True)
        sd[0].wait()
        sd[1].wait()
        return carry

    lax.fori_loop(0, nchunks // GROUP, group_body, 0)


def _seg_body(do_deg, *refs):
    if do_deg:
        (table, src2d, dst2d, zrows, eye, out,
         sidx, didx, rows0, rows1, gsem0, gsem1, ssem0, ssem1,
         acc, ohsp) = refs
    else:
        (table, src2d, dst2d, zrows, out,
         sidx, didx, rows0, rows1, gsem0, gsem1, ssem0, ssem1, acc) = refs

    c = lax.axis_index("c")
    s = lax.axis_index("s")
    wid = c * NS + s
    rows = (rows0, rows1)
    gsems = (gsem0, gsem1)
    ssems = (ssem0, ssem1)

    pltpu.sync_copy(zrows, acc.at[pl.ds(s * ROWS_PER_TILE, ROWS_PER_TILE)])
    if do_deg:
        @pl.when(s == 0)
        def _():
            pltpu.sync_copy(eye, ohsp)

    plsc.subcore_barrier()

    _run_phase(table, src2d, dst2d, acc, sidx, didx, rows, gsems, ssems,
               wid * PHASE_CPT, PHASE_CPT)
    if do_deg:
        _run_phase(ohsp, src2d, dst2d, acc, sidx, didx, rows, gsems, ssems,
                   PHASE_ROWS + wid * PHASE_CPT, PHASE_CPT)

    plsc.subcore_barrier()

    sl = pl.ds(s * ROWS_PER_TILE, ROWS_PER_TILE)
    pltpu.sync_copy(acc.at[sl], out.at[c].at[sl])


def _make_seg_kernel(do_deg):
    scratch = [
        pltpu.VMEM((GROUP, CHUNK), jnp.int32),
        pltpu.VMEM((GROUP, CHUNK), jnp.int32),
        pltpu.VMEM((CHUNK, D), jnp.float32),
        pltpu.VMEM((CHUNK, D), jnp.float32),
        pltpu.SemaphoreType.DMA,
        pltpu.SemaphoreType.DMA,
        pltpu.SemaphoreType.DMA,
        pltpu.SemaphoreType.DMA,
        pltpu.VMEM_SHARED((NACC, D), jnp.float32),
    ]
    if do_deg:
        scratch.append(pltpu.VMEM_SHARED((128, D), jnp.float32))
    return pl.kernel(
        functools.partial(_seg_body, do_deg),
        out_type=jax.ShapeDtypeStruct((NC, NACC, D), jnp.float32),
        mesh=plsc.VectorSubcoreMesh(core_axis_name="c", subcore_axis_name="s"),
        scratch_types=scratch,
    )


_seg_deg = _make_seg_kernel(True)
_seg = _make_seg_kernel(False)


def _selu(v):
    return _SELU_SCALE * jnp.where(v > 0, v, _SELU_ALPHA * (jnp.exp(v) - 1.0))


def _tc_body(activation, p_ref, deg_ref, w_ref, b_ref, o_ref):
    p = p_ref[0] + p_ref[1]
    dinv = 1.0 / jnp.maximum(deg_ref[...], 1.0)
    v = jnp.dot(p * dinv, w_ref[...], preferred_element_type=jnp.float32)
    v = v + b_ref[...]
    if activation:
        v = _selu(v)
    o_ref[...] = v


RB = 1024


def _make_tc_kernel(activation):
    return pl.pallas_call(
        functools.partial(_tc_body, activation),
        grid=(NP // RB,),
        in_specs=[
            pl.BlockSpec((NC, RB, D), lambda i: (0, i, 0)),
            pl.BlockSpec((RB, 1), lambda i: (i, 0)),
            pl.BlockSpec((D, D), lambda i: (0, 0)),
            pl.BlockSpec((1, D), lambda i: (0, 0)),
        ],
        out_specs=pl.BlockSpec((RB, D), lambda i: (i, 0)),
        out_shape=jax.ShapeDtypeStruct((NP, D), jnp.float32),
    )


_tc_act = _make_tc_kernel(True)
_tc_lin = _make_tc_kernel(False)


def kernel(x, edge_index, W1, b1, W2, b2):
    ei = edge_index.astype(jnp.int32)
    src, dst = ei[0], ei[1]

    oh_src = dst & 127
    deg_dst = NP + lax.shift_right_logical(dst, 7)
    padn = PHASE_E - N_EDGES
    spad = jnp.zeros((padn,), jnp.int32)
    dpad = jnp.full((padn,), NACC - 1, jnp.int32)
    s1 = jnp.concatenate([src, spad, oh_src, spad])
    d1 = jnp.concatenate([dst, dpad, deg_dst, dpad])
    src2d_1 = s1.reshape(-1, CHUNK)
    dst2d_1 = d1.reshape(-1, CHUNK)

    s2 = jnp.concatenate([src, spad])
    d2 = jnp.concatenate([dst, dpad])
    src2d_2 = s2.reshape(-1, CHUNK)
    dst2d_2 = d2.reshape(-1, CHUNK)

    eye = jnp.eye(D, dtype=jnp.float32)
    z = jnp.zeros((ROWS_PER_TILE, D), jnp.float32)

    p = _seg_deg(x, src2d_1, dst2d_1, z, eye)
    deg = (p[0, NP:NP + 80] + p[1, NP:NP + 80]).reshape(NP, 1)
    h = _tc_act(p, deg, W1, b1.reshape(1, D))
    q = _seg(h, src2d_2, dst2d_2, z)
    out = _tc_lin(q, deg, W2, b2.reshape(1, D))
    return out[:N_NODES]
